# Initial kernel scaffold; baseline (speedup 1.0000x reference)
import jax
import jax.numpy as jnp
from jax.experimental import pallas as pl


def kernel(x, syndrome):
    raise NotImplementedError("write your pallas kernel here")


# speedup vs baseline: 38.1090x; 38.1090x over previous
import numpy as np
import jax
import jax.numpy as jnp
from jax.experimental import pallas as pl

_L = 16
_B = 64
_N = _L ** 3
_BB = 8


def _np_cond_matrix(roll_first: bool) -> np.ndarray:
    m = np.zeros((_L, _L), np.int64)
    for s in range(_L):
        v = np.zeros(_L, np.int64)
        v[s] = 1
        if roll_first:
            v = np.roll(v, 1)
        w = np.roll(np.flip(v), 1)
        c = np.cumsum(w)
        m[:, s] = np.roll(c, 1)
    return m


def _np_perm_tables() -> np.ndarray:
    def swap(t):
        s = t.shape
        return t.transpose(3, 2, 1, 0).reshape(s)[[0, 6, 2, 4, 3, 5, 1, 7]]

    base = np.arange(64).reshape(8, 2, 2, 2)
    tbl = np.zeros((64, 64), np.int32)
    for code in range(64):
        v = base.copy()
        for a in range(3):
            if (code >> (2 * a)) & 1:
                v = swap(np.roll(swap(v), 1, axis=-3 + a))
            if (code >> (2 * a + 1)) & 1:
                v = np.roll(v, 1, axis=-1 - a)
        tbl[code] = v.reshape(64)
    return tbl


_MC = _np_cond_matrix(False).astype(np.float32)
_MCR = _np_cond_matrix(True).astype(np.float32)
_TBL = _np_perm_tables()
_WC = (np.transpose(
    (_TBL[:, None, :] == np.arange(64)[None, :, None]).astype(np.float32),
    (0, 1, 2)).reshape(64 * 64, 64) / float(_N)).astype(np.float32)


def _codes_body(zs_ref, x0_ref, x1_ref, x2_ref, mz_ref, mx0_ref, out_ref):
    f32 = jnp.float32
    l = _L
    zs = zs_ref[...].astype(f32)
    r0 = jnp.sum(zs, axis=1)
    r1 = jnp.sum(zs, axis=2)
    r2 = jnp.sum(zs, axis=3)
    mz = mz_ref[...]
    mx0 = mx0_ref[...]

    def cond2(r):
        s = jax.lax.dot_general(r.reshape(_B * l, l), mz,
                                (((1,), (1,)), ((), ())),
                                preferred_element_type=f32)
        return (s.astype(jnp.int32) & 1).reshape(_B, l, l)

    fz0 = cond2(r0)
    fz1 = cond2(r1)
    fz2 = cond2(r2)

    q0 = jnp.sum(x0_ref[...].astype(f32), axis=(1, 2))
    q1 = jnp.sum(x1_ref[...].astype(f32), axis=(2, 3))
    q2 = jnp.sum(x2_ref[...].astype(f32), axis=(1, 3))

    def cond1(q, m):
        s = jax.lax.dot_general(q, m, (((1,), (1,)), ((), ())),
                                preferred_element_type=f32)
        return s.astype(jnp.int32) & 1

    fx0 = cond1(q0, mx0)
    fx1 = cond1(q1, mz)
    fx2 = cond1(q2, mz)

    d0 = jnp.swapaxes(fz0, 1, 2)
    d2 = jnp.swapaxes(fz2, 1, 2)
    code = (d0[:, None, :, :]
            + 2 * fx0[:, None, None, :]
            + 4 * fz1[:, :, None, :]
            + 8 * fx1[:, :, None, None]
            + 16 * d2[:, :, :, None]
            + 32 * fx2[:, None, :, None])
    out_ref[...] = code.astype(jnp.int32)


def _bucket_body(code_ref, x_ref, o_ref):
    giota = jax.lax.broadcasted_iota(jnp.int32, (64, _N), 0)
    for b in range(_BB):
        c2 = code_ref[pl.ds(b, 1), :]
        at = (jnp.broadcast_to(c2, (64, _N)) == giota).astype(jnp.float32)
        xb = x_ref[b]
        s = jax.lax.dot_general(at, xb, (((1,), (0,)), ((), ())),
                                preferred_element_type=jnp.float32)
        o_ref[b] = s


def _combine_body(s_ref, w_ref, o_ref):
    o_ref[...] = jax.lax.dot_general(
        s_ref[...], w_ref[...], (((1,), (0,)), ((), ())),
        preferred_element_type=jnp.float32)


def kernel(x, syndrome):
    l, b, n = _L, _B, _N
    zs = syndrome[:, :n].reshape(b, l, l, l)
    xs0 = syndrome[:, n:2 * n].reshape(b, l, l, l)
    xs1 = syndrome[:, 2 * n:3 * n].reshape(b, l, l, l)
    xs2 = syndrome[:, 3 * n:].reshape(b, l, l, l)

    code4 = pl.pallas_call(
        _codes_body,
        out_shape=jax.ShapeDtypeStruct((b, l, l, l), jnp.int32),
    )(zs, xs0, xs1, xs2, jnp.asarray(_MC), jnp.asarray(_MCR))
    code = code4.reshape(b, n)

    x2 = x.reshape(b, n, 64)
    s = pl.pallas_call(
        _bucket_body,
        grid=(b // _BB,),
        in_specs=[
            pl.BlockSpec((_BB, n), lambda i: (i, 0)),
            pl.BlockSpec((_BB, n, 64), lambda i: (i, 0, 0)),
        ],
        out_specs=pl.BlockSpec((_BB, 64, 64), lambda i: (i, 0, 0)),
        out_shape=jax.ShapeDtypeStruct((b, 64, 64), jnp.float32),
    )(code, x2)

    out = pl.pallas_call(
        _combine_body,
        out_shape=jax.ShapeDtypeStruct((b, 64), jnp.float32),
    )(s.reshape(b, 64 * 64), jnp.asarray(_WC))
    return out.reshape(b, 8, 2, 2, 2)


# speedup vs baseline: 64.9815x; 1.7051x over previous
import numpy as np
import jax
import jax.numpy as jnp
from jax.experimental import pallas as pl

_L = 16
_B = 64
_N = _L ** 3
_H = _N // 2
_BB = 8


def _np_cond_matrix(roll_first: bool) -> np.ndarray:
    m = np.zeros((_L, _L), np.int64)
    for s in range(_L):
        v = np.zeros(_L, np.int64)
        v[s] = 1
        if roll_first:
            v = np.roll(v, 1)
        w = np.roll(np.flip(v), 1)
        m[:, s] = np.roll(np.cumsum(w), 1)
    return m


def _np_perm_tables() -> np.ndarray:
    def swap(t):
        s = t.shape
        return t.transpose(3, 2, 1, 0).reshape(s)[[0, 6, 2, 4, 3, 5, 1, 7]]

    base = np.arange(64).reshape(8, 2, 2, 2)
    tbl = np.zeros((64, 64), np.int32)
    for code in range(64):
        v = base.copy()
        for a in range(3):
            if (code >> (2 * a)) & 1:
                v = swap(np.roll(swap(v), 1, axis=-3 + a))
            if (code >> (2 * a + 1)) & 1:
                v = np.roll(v, 1, axis=-1 - a)
        tbl[code] = v.reshape(64)
    return tbl


def _np_code_mats():
    mc = _np_cond_matrix(False)
    mcr = _np_cond_matrix(True)
    p = np.arange(_N)
    pi, pj, pk = p // 256, (p // 16) % 16, p % 16
    g = np.arange(256) // 16
    t = np.arange(256) % 16
    r16 = np.arange(16)
    reduce_mats = [
        ((pj[:, None] == g[None, :]) * mc[t[None, :], pk[:, None]]),
        ((pi[:, None] == g[None, :]) * mc[t[None, :], pk[:, None]]),
        ((pi[:, None] == g[None, :]) * mc[t[None, :], pj[:, None]]),
        mcr[r16[None, :], pk[:, None]],
        mc[r16[None, :], pi[:, None]],
        mc[r16[None, :], pj[:, None]],
    ]
    expand_mats = [
        ((g[:, None] == pk[None, :]) & (t[:, None] == pj[None, :])) * 1,
        ((g[:, None] == pi[None, :]) & (t[:, None] == pk[None, :])) * 4,
        ((g[:, None] == pj[None, :]) & (t[:, None] == pi[None, :])) * 16,
        (r16[:, None] == pk[None, :]) * 2,
        (r16[:, None] == pi[None, :]) * 8,
        (r16[:, None] == pj[None, :]) * 32,
    ]
    newcol = (p % 2) * _H + p // 2
    expand2 = []
    for e in expand_mats:
        e2 = np.zeros_like(e)
        e2[:, newcol] = e
        expand2.append(e2)
    return ([np.asarray(m, np.float32) for m in reduce_mats],
            [np.asarray(m, np.float32) for m in expand2])


_REDUCE_MATS, _EXPAND_MATS = _np_code_mats()
_TBL = _np_perm_tables()

_WC2 = np.zeros((128 * 128, 64), np.float32)
_sel = (_TBL[:, None, :] == np.arange(64)[None, :, None]).astype(np.float32)
for _g in range(64):
    for _ci in range(64):
        _w = _sel[_g, _ci] / float(_N)
        _WC2[_g * 128 + _ci] += _w
        _WC2[(64 + _g) * 128 + 64 + _ci] += _w


def _codes_body(syn_ref, cz0, cz1, cz2, cx0, cx1, cx2,
                ez0, ez1, ez2, ex0, ex1, ex2, oe_ref, oo_ref):
    bf = jnp.bfloat16
    f32 = jnp.float32
    s = syn_ref[...].astype(bf)
    parts = [s[:, :_N], s[:, _N:2 * _N], s[:, 2 * _N:3 * _N], s[:, 3 * _N:]]

    def mm(a, b_ref):
        return jax.lax.dot_general(a, b_ref[...], (((1,), (0,)), ((), ())),
                                   preferred_element_type=f32)

    def bits(pre):
        return (pre.astype(jnp.int32) & 1).astype(bf)

    code = (mm(bits(mm(parts[0], cz0)), ez0)
            + mm(bits(mm(parts[0], cz1)), ez1)
            + mm(bits(mm(parts[0], cz2)), ez2)
            + mm(bits(mm(parts[1], cx0)), ex0)
            + mm(bits(mm(parts[2], cx1)), ex1)
            + mm(bits(mm(parts[3], cx2)), ex2)).astype(jnp.int32)
    oe_ref[...] = code[:, :_H]
    oo_ref[...] = code[:, _H:]


def _bucket_body(ce_ref, co_ref, x_ref, o_ref):
    giota = jax.lax.broadcasted_iota(jnp.int32, (64, _H), 0)
    for b in range(_BB):
        ce = ce_ref[pl.ds(b, 1), :]
        co = co_ref[pl.ds(b, 1), :]
        ate = (jnp.broadcast_to(ce, (64, _H)) == giota).astype(jnp.float32)
        ato = (jnp.broadcast_to(co, (64, _H)) == giota).astype(jnp.float32)
        xb = x_ref[b]
        se = jax.lax.dot_general(ate, xb, (((1,), (0,)), ((), ())),
                                 preferred_element_type=jnp.float32)
        so = jax.lax.dot_general(ato, xb, (((1,), (0,)), ((), ())),
                                 preferred_element_type=jnp.float32)
        o_ref[b, pl.ds(0, 64), :] = se
        o_ref[b, pl.ds(64, 64), :] = so


def _combine_body(s_ref, w_ref, o_ref):
    o_ref[...] = jax.lax.dot_general(
        s_ref[...], w_ref[...], (((1,), (0,)), ((), ())),
        preferred_element_type=jnp.float32)


def kernel(x, syndrome):
    b, n, h = _B, _N, _H
    bf = jnp.bfloat16
    consts = ([jnp.asarray(m, bf) for m in _REDUCE_MATS]
              + [jnp.asarray(m, bf) for m in _EXPAND_MATS])
    code_e, code_o = pl.pallas_call(
        _codes_body,
        out_shape=[jax.ShapeDtypeStruct((b, h), jnp.int32),
                   jax.ShapeDtypeStruct((b, h), jnp.int32)],
    )(syndrome, *consts)

    x2 = x.reshape(b, h, 128)
    s = pl.pallas_call(
        _bucket_body,
        grid=(b // _BB,),
        in_specs=[
            pl.BlockSpec((_BB, h), lambda i: (i, 0)),
            pl.BlockSpec((_BB, h), lambda i: (i, 0)),
            pl.BlockSpec((_BB, h, 128), lambda i: (i, 0, 0)),
        ],
        out_specs=pl.BlockSpec((_BB, 128, 128), lambda i: (i, 0, 0)),
        out_shape=jax.ShapeDtypeStruct((b, 128, 128), jnp.float32),
    )(code_e, code_o, x2)

    out = pl.pallas_call(
        _combine_body,
        out_shape=jax.ShapeDtypeStruct((b, 64), jnp.float32),
    )(s.reshape(b, 128 * 128), jnp.asarray(_WC2))
    return out.reshape(b, 8, 2, 2, 2)
